# Initial kernel scaffold; baseline (speedup 1.0000x reference)
import jax
import jax.numpy as jnp
from jax.experimental import pallas as pl


def kernel(x, gate_w, wg, wu, wd):
    raise NotImplementedError("write your pallas kernel here")


# speedup vs baseline: 1.1500x; 1.1500x over previous
import functools

import jax
import jax.numpy as jnp
from jax.experimental import pallas as pl
from jax.experimental.pallas import tpu as pltpu

N = 2048
C = 768
E = 8
H = 2048
CAPACITY = 640


def _router_body(x_ref, gw_ref, w_ref, aux_ref, z_ref):
    xf = x_ref[...]
    gw = gw_ref[...]
    logits = jax.lax.dot_general(
        xf, gw, (((1,), (1,)), ((), ())), preferred_element_type=jnp.float32)
    m = jnp.max(logits, axis=1, keepdims=True)
    ex = jnp.exp(logits - m)
    gates = ex / jnp.sum(ex, axis=1, keepdims=True)

    lane = jax.lax.broadcasted_iota(jnp.int32, (N, E), 1)
    top1_v = jnp.max(gates, axis=1, keepdims=True)
    top1_i = jnp.min(jnp.where(gates == top1_v, lane, E), axis=1,
                     keepdims=True)
    masked = jnp.where(lane == top1_i, -jnp.inf, gates)
    top2_v = jnp.max(masked, axis=1, keepdims=True)
    top2_i = jnp.min(jnp.where(masked == top2_v, lane, E), axis=1,
                     keepdims=True)

    me = jnp.sum(gates, axis=0, keepdims=True) * (1.0 / N)
    onehot1 = (lane == top1_i).astype(jnp.float32)
    ce = jnp.sum(onehot1, axis=0, keepdims=True) * (1.0 / N)
    aux_ref[...] = E * jnp.sum(me * ce, axis=1, keepdims=True)
    zrow = jnp.sum(logits * logits, axis=0, keepdims=True)
    z_ref[...] = jnp.sum(zrow, axis=1, keepdims=True) * (1.0 / (N * E))

    r = jax.lax.broadcasted_iota(jnp.int32, (N, N), 0)
    ccol = jax.lax.broadcasted_iota(jnp.int32, (N, N), 1)
    tri = (r >= ccol).astype(jnp.bfloat16)
    onehot2 = (lane == top2_i).astype(jnp.float32)
    ranks1 = jax.lax.dot_general(
        tri, onehot1.astype(jnp.bfloat16), (((1,), (0,)), ((), ())),
        preferred_element_type=jnp.float32)
    ranks2 = jax.lax.dot_general(
        tri, onehot2.astype(jnp.bfloat16), (((1,), (0,)), ((), ())),
        preferred_element_type=jnp.float32)
    keep1 = (onehot1 > 0) & (ranks1 <= CAPACITY)
    keep2 = (onehot2 > 0) & (ranks2 <= CAPACITY)
    w1 = jnp.where(keep1, top1_v, 0.0)
    w2 = jnp.where(keep2, top2_v, 0.0)
    w_ref[...] = (w1 * onehot1 + w2 * onehot2).astype(jnp.float32)


def _router(xf, gate_w):
    return pl.pallas_call(
        _router_body,
        out_shape=(
            jax.ShapeDtypeStruct((N, E), jnp.float32),
            jax.ShapeDtypeStruct((1, 1), jnp.float32),
            jax.ShapeDtypeStruct((1, 1), jnp.float32),
        ),
    )(xf, gate_w)


_RB = 512


def _expert_body(x_ref, w_ref, wg_ref, wu_ref, wd_ref, out_ref):
    e = pl.program_id(0)
    rb = pl.program_id(1)
    xb = x_ref[...]
    hg = jax.lax.dot_general(
        xb, wg_ref[0], (((1,), (1,)), ((), ())),
        preferred_element_type=jnp.float32)
    hu = jax.lax.dot_general(
        xb, wu_ref[0], (((1,), (1,)), ((), ())),
        preferred_element_type=jnp.float32)
    silu = hg / (1.0 + jnp.exp(-hg))
    hidden = (silu * hu).astype(jnp.bfloat16)
    y = jax.lax.dot_general(
        hidden, wd_ref[0], (((1,), (1,)), ((), ())),
        preferred_element_type=jnp.float32)
    lane = jax.lax.broadcasted_iota(jnp.int32, (_RB, E), 1)
    wcol = jnp.sum(jnp.where(lane == e, w_ref[...], 0.0), axis=1,
                   keepdims=True)
    contrib = wcol * y
    sl = pl.ds(rb * _RB, _RB)

    @pl.when(e == 0)
    def _():
        out_ref[sl, :] = contrib

    @pl.when(e != 0)
    def _():
        out_ref[sl, :] = out_ref[sl, :] + contrib


def _experts(x16, w, wg16, wu16, wd16):
    grid = (E, N // _RB)
    return pl.pallas_call(
        _expert_body,
        grid=grid,
        in_specs=[
            pl.BlockSpec((_RB, C), lambda e, rb: (rb, 0)),
            pl.BlockSpec((_RB, E), lambda e, rb: (rb, 0)),
            pl.BlockSpec((1, H, C), lambda e, rb: (e, 0, 0)),
            pl.BlockSpec((1, H, C), lambda e, rb: (e, 0, 0)),
            pl.BlockSpec((1, C, H), lambda e, rb: (e, 0, 0)),
        ],
        out_specs=pl.BlockSpec((N, C), lambda e, rb: (0, 0)),
        out_shape=jax.ShapeDtypeStruct((N, C), jnp.float32),
    )(x16, w, wg16, wu16, wd16)


def kernel(x, gate_w, wg, wu, wd):
    b, t, c = x.shape
    xf = x.reshape(b * t, c)
    w, aux, z = _router(xf, gate_w)
    out = _experts(xf.astype(jnp.bfloat16), w,
                   wg.astype(jnp.bfloat16), wu.astype(jnp.bfloat16),
                   wd.astype(jnp.bfloat16))
    return out.reshape(b, t, c), aux[0, 0], z[0, 0]
